# Initial kernel scaffold; baseline (speedup 1.0000x reference)
import jax
import jax.numpy as jnp
from jax.experimental import pallas as pl


def kernel(x, edge_index, edge_attr, We, be, W1, W2, gamma, beta):
    raise NotImplementedError("write your pallas kernel here")


# speedup vs baseline: 2.1010x; 2.1010x over previous
import functools

import jax
import jax.numpy as jnp
from jax import lax
from jax.experimental import pallas as pl
from jax.experimental.pallas import tpu as pltpu
from jax.experimental.pallas import tpu_sc as plsc

N = 10000
E = 320000
D = 128
DE = 16
H = 128

NC = 2
NS = 16
NW = NC * NS
EPT = E // NW
BLK = 80
NBLK = EPT // BLK
RPT = 624
RTAIL = N - NS * RPT


def _yz_body(x_ref, wx_ref, y_ref):
    y_ref[...] = jnp.dot(x_ref[...], wx_ref[...],
                         preferred_element_type=jnp.float32)


def _tc_y(x, wxt):
    return pl.pallas_call(
        _yz_body,
        out_shape=jax.ShapeDtypeStruct((N, H), jnp.float32),
    )(x, wxt)


def _z_body(e_ref, we_ref, b_ref, z_ref):
    z_ref[...] = jnp.dot(e_ref[...], we_ref[...],
                         preferred_element_type=jnp.float32) + b_ref[...]


def _tc_z(edge_attr, wet, b2d):
    be_blk = 8000
    return pl.pallas_call(
        _z_body,
        grid=(E // be_blk,),
        in_specs=[
            pl.BlockSpec((be_blk, DE), lambda i: (i, 0)),
            pl.BlockSpec((DE, H), lambda i: (0, 0)),
            pl.BlockSpec((1, H), lambda i: (0, 0)),
        ],
        out_specs=pl.BlockSpec((be_blk, H), lambda i: (i, 0)),
        out_shape=jax.ShapeDtypeStruct((E, H), jnp.float32),
    )(edge_attr, wet, b2d)


def _sc_body(src_hbm, dst_hbm, y_hbm, z_hbm, zero_hbm, out_hbm,
             idx_s, idx_d, yrows, zblk, agg_sh, sem):
    c = lax.axis_index("c")
    s = lax.axis_index("s")
    wid = s * NC + c

    row0 = pl.multiple_of(s * RPT, 8)
    pltpu.sync_copy(zero_hbm.at[pl.ds(row0, RPT)],
                    agg_sh.at[pl.ds(row0, RPT)])

    @pl.when(s == 0)
    def _():
        pltpu.sync_copy(zero_hbm.at[pl.ds(NS * RPT, RTAIL)],
                        agg_sh.at[pl.ds(NS * RPT, RTAIL)])
    plsc.subcore_barrier()

    base0 = wid * EPT

    def block(i, carry):
        base = pl.multiple_of(base0 + i * BLK, 8)
        pltpu.sync_copy(src_hbm.at[pl.ds(base, BLK)], idx_s)
        pltpu.sync_copy(dst_hbm.at[pl.ds(base, BLK)], idx_d)
        gcp = pltpu.async_copy(y_hbm.at[idx_s], yrows, sem)
        pltpu.sync_copy(z_hbm.at[pl.ds(base, BLK)], zblk)
        gcp.wait()

        def row(r, carry2):
            for j in range(H // 16):
                sl = pl.ds(j * 16, 16)
                v = yrows[r, sl] + zblk[r, sl]
                zblk[r, sl] = jnp.maximum(v, 0.0)
            return carry2
        lax.fori_loop(0, BLK, row, 0, unroll=2)

        pltpu.sync_copy(zblk, agg_sh.at[idx_d], add=True)
        return carry

    lax.fori_loop(0, NBLK, block, 0)
    plsc.subcore_barrier()

    pltpu.sync_copy(agg_sh.at[pl.ds(row0, RPT)],
                    out_hbm.at[c, pl.ds(row0, RPT)])

    @pl.when(s == 0)
    def _():
        pltpu.sync_copy(agg_sh.at[pl.ds(NS * RPT, RTAIL)],
                        out_hbm.at[c, pl.ds(NS * RPT, RTAIL)])


@functools.cache
def _sc_agg():
    return pl.kernel(
        _sc_body,
        out_type=jax.ShapeDtypeStruct((NC, N, H), jnp.float32),
        mesh=plsc.VectorSubcoreMesh(core_axis_name="c", subcore_axis_name="s",
                                    num_cores=NC, num_subcores=NS),
        scratch_types=[
            pltpu.VMEM((BLK,), jnp.int32),
            pltpu.VMEM((BLK,), jnp.int32),
            pltpu.VMEM((BLK, H), jnp.float32),
            pltpu.VMEM((BLK, H), jnp.float32),
            pltpu.VMEM_SHARED((N, H), jnp.float32),
            pltpu.SemaphoreType.DMA,
        ],
    )


def _final_body(x_ref, agg_ref, w1_ref, w2_ref, g_ref, b_ref, out_ref):
    pre = x_ref[...] + agg_ref[0] + agg_ref[1]
    h1 = jnp.dot(pre, w1_ref[...], preferred_element_type=jnp.float32)
    mean = jnp.mean(h1, axis=0, keepdims=True)
    var = jnp.mean((h1 - mean) ** 2, axis=0, keepdims=True)
    hn = (h1 - mean) * jax.lax.rsqrt(var + 1e-5) * g_ref[...] + b_ref[...]
    h = jnp.maximum(hn, 0.0)
    out_ref[...] = jnp.dot(h, w2_ref[...], preferred_element_type=jnp.float32)


def _tc_final(x, agg, w1t, w2t, g2d, b2d):
    return pl.pallas_call(
        _final_body,
        out_shape=jax.ShapeDtypeStruct((N, D), jnp.float32),
    )(x, agg, w1t, w2t, g2d, b2d)


def kernel(x, edge_index, edge_attr, We, be, W1, W2, gamma, beta):
    src = edge_index[0]
    dst = edge_index[1]
    wxt = We[:, :D].T
    wet = We[:, D:].T
    y = _tc_y(x, wxt)
    z = _tc_z(edge_attr, wet, be.reshape(1, H))
    zero = jnp.zeros((N, H), jnp.float32)
    agg = _sc_agg()(src, dst, y, z, zero)
    return _tc_final(x, agg, W1.T, W2.T,
                     gamma.reshape(1, H), beta.reshape(1, H))


# speedup vs baseline: 2.9093x; 1.3847x over previous
import functools

import jax
import jax.numpy as jnp
from jax import lax
from jax.experimental import pallas as pl
from jax.experimental.pallas import tpu as pltpu
from jax.experimental.pallas import tpu_sc as plsc

N = 10000
E = 320000
D = 128
DE = 16
H = 128

NC = 2
NS = 16
NW = NC * NS
EPT = E // NW
BLK = 40
NBLK = EPT // BLK
RPT = 624
RTAIL = N - NS * RPT


def _yz_body(x_ref, wx_ref, y_ref):
    y_ref[...] = jnp.dot(x_ref[...], wx_ref[...],
                         preferred_element_type=jnp.float32)


def _tc_y(x, wxt):
    return pl.pallas_call(
        _yz_body,
        out_shape=jax.ShapeDtypeStruct((N, H), jnp.float32),
    )(x, wxt)


def _z_body(e_ref, we_ref, b_ref, z_ref):
    z_ref[...] = jnp.dot(e_ref[...], we_ref[...],
                         preferred_element_type=jnp.float32) + b_ref[...]


def _tc_z(edge_attr, wet, b2d):
    be_blk = 8000
    return pl.pallas_call(
        _z_body,
        grid=(E // be_blk,),
        in_specs=[
            pl.BlockSpec((be_blk, DE), lambda i: (i, 0)),
            pl.BlockSpec((DE, H), lambda i: (0, 0)),
            pl.BlockSpec((1, H), lambda i: (0, 0)),
        ],
        out_specs=pl.BlockSpec((be_blk, H), lambda i: (i, 0)),
        out_shape=jax.ShapeDtypeStruct((E, H), jnp.float32),
    )(edge_attr, wet, b2d)


def _sc_body(src_hbm, dst_hbm, y_hbm, z_hbm, zero_hbm, out_hbm,
             idxs_v, idxd_v, yrow2, zbuf2, msg2, agg_sh,
             sg0, sg1, sz0, sz1, ss0, ss1):
    c = lax.axis_index("c")
    s = lax.axis_index("s")
    wid = s * NC + c
    base0 = wid * EPT
    sg = (sg0, sg1)
    sz = (sz0, sz1)
    ss = (ss0, ss1)

    pltpu.sync_copy(src_hbm.at[pl.ds(base0, EPT)], idxs_v)
    pltpu.sync_copy(dst_hbm.at[pl.ds(base0, EPT)], idxd_v)
    row0 = pl.multiple_of(s * RPT, 8)
    pltpu.sync_copy(zero_hbm.at[pl.ds(row0, RPT)],
                    agg_sh.at[pl.ds(row0, RPT)])

    @pl.when(s == 0)
    def _():
        pltpu.sync_copy(zero_hbm.at[pl.ds(NS * RPT, RTAIL)],
                        agg_sh.at[pl.ds(NS * RPT, RTAIL)])
    plsc.subcore_barrier()

    def isl(i):
        return pl.ds(pl.multiple_of(i * BLK, 8), BLK)

    def g_desc(i, b):
        return pltpu.make_async_copy(y_hbm.at[idxs_v.at[isl(i)]],
                                     yrow2.at[b], sg[b])

    def z_desc(i, b):
        return pltpu.make_async_copy(
            z_hbm.at[pl.ds(pl.multiple_of(base0 + i * BLK, 8), BLK)],
            zbuf2.at[b], sz[b])

    def s_desc(i, b):
        return pltpu.make_async_copy(msg2.at[b],
                                     agg_sh.at[idxd_v.at[isl(i)]], ss[b])

    def compute(b):
        def row(r, carry):
            for j in range(H // 16):
                sl = pl.ds(j * 16, 16)
                v = yrow2[b, r, sl] + zbuf2[b, r, sl]
                msg2[b, r, sl] = jnp.maximum(v, 0.0)
            return carry
        lax.fori_loop(0, BLK, row, 0, unroll=2)

    for b in (0, 1):
        g_desc(b, b).start()
        z_desc(b, b).start()

    def pair(k, carry):
        for b in (0, 1):
            i = 2 * k + b

            @pl.when(k >= 1)
            def _():
                s_desc(i - 2, b).wait()

            g_desc(i, b).wait()
            z_desc(i, b).wait()
            compute(b)
            s_desc(i, b).start(add=True)

            @pl.when(i + 2 < NBLK)
            def _():
                g_desc(i + 2, b).start()
                z_desc(i + 2, b).start()
        return carry
    lax.fori_loop(0, NBLK // 2, pair, 0)

    s_desc(NBLK - 2, 0).wait()
    s_desc(NBLK - 1, 1).wait()
    plsc.subcore_barrier()

    pltpu.sync_copy(agg_sh.at[pl.ds(row0, RPT)],
                    out_hbm.at[c, pl.ds(row0, RPT)])

    @pl.when(s == 0)
    def _():
        pltpu.sync_copy(agg_sh.at[pl.ds(NS * RPT, RTAIL)],
                        out_hbm.at[c, pl.ds(NS * RPT, RTAIL)])


@functools.cache
def _sc_agg():
    return pl.kernel(
        _sc_body,
        out_type=jax.ShapeDtypeStruct((NC, N, H), jnp.float32),
        mesh=plsc.VectorSubcoreMesh(core_axis_name="c", subcore_axis_name="s",
                                    num_cores=NC, num_subcores=NS),
        scratch_types=[
            pltpu.VMEM((EPT,), jnp.int32),
            pltpu.VMEM((EPT,), jnp.int32),
            pltpu.VMEM((2, BLK, H), jnp.float32),
            pltpu.VMEM((2, BLK, H), jnp.float32),
            pltpu.VMEM((2, BLK, H), jnp.float32),
            pltpu.VMEM_SHARED((N, H), jnp.float32),
            pltpu.SemaphoreType.DMA,
            pltpu.SemaphoreType.DMA,
            pltpu.SemaphoreType.DMA,
            pltpu.SemaphoreType.DMA,
            pltpu.SemaphoreType.DMA,
            pltpu.SemaphoreType.DMA,
        ],
    )


def _final_body(x_ref, agg_ref, w1_ref, w2_ref, g_ref, b_ref, out_ref):
    pre = x_ref[...] + agg_ref[0] + agg_ref[1]
    h1 = jnp.dot(pre, w1_ref[...], preferred_element_type=jnp.float32)
    mean = jnp.mean(h1, axis=0, keepdims=True)
    var = jnp.mean((h1 - mean) ** 2, axis=0, keepdims=True)
    hn = (h1 - mean) * jax.lax.rsqrt(var + 1e-5) * g_ref[...] + b_ref[...]
    h = jnp.maximum(hn, 0.0)
    out_ref[...] = jnp.dot(h, w2_ref[...], preferred_element_type=jnp.float32)


def _tc_final(x, agg, w1t, w2t, g2d, b2d):
    return pl.pallas_call(
        _final_body,
        out_shape=jax.ShapeDtypeStruct((N, D), jnp.float32),
    )(x, agg, w1t, w2t, g2d, b2d)


def kernel(x, edge_index, edge_attr, We, be, W1, W2, gamma, beta):
    src = edge_index[0]
    dst = edge_index[1]
    wxt = We[:, :D].T
    wet = We[:, D:].T
    y = _tc_y(x, wxt)
    z = _tc_z(edge_attr, wet, be.reshape(1, H))
    zero = jnp.zeros((N, H), jnp.float32)
    agg = _sc_agg()(src, dst, y, z, zero)
    return _tc_final(x, agg, W1.T, W2.T,
                     gamma.reshape(1, H), beta.reshape(1, H))
